# Initial kernel scaffold; baseline (speedup 1.0000x reference)
import functools

import jax
import jax.numpy as jnp
from jax import lax
from jax.experimental import pallas as pl
from jax.experimental.pallas import tpu as pltpu
from jax.experimental.pallas import tpu_sc as plsc

_L = 16
f32 = jnp.float32
i32 = jnp.int32


def _iota16():
    return lax.iota(i32, _L)


def _splat_i(v):
    return jnp.zeros((_L,), i32) + v


@functools.partial(jax.jit, static_argnames=("S", "n_rays"))
def _sc_gather(ox, oy, oz, dx, dy, dz, ray, ts, te, *, S, n_rays):
    NW = 32
    CH = S // NW
    HB = CH // 2
    mesh = plsc.VectorSubcoreMesh(core_axis_name="c", subcore_axis_name="s")
    out = [jax.ShapeDtypeStruct((S,), f32) for _ in range(7)]
    scratch = (
        [pltpu.VMEM((n_rays,), f32) for _ in range(6)]
        + [pltpu.VMEM((CH,), i32), pltpu.VMEM((CH,), f32), pltpu.VMEM((CH,), f32)]
        + [pltpu.VMEM((HB,), f32) for _ in range(7)]
    )

    @functools.partial(pl.kernel, mesh=mesh, out_type=out, scratch_types=scratch)
    def k(ox_h, oy_h, oz_h, dx_h, dy_h, dz_h, ray_h, ts_h, te_h,
          px_h, py_h, pz_h, gx_h, gy_h, gz_h, dl_h,
          tox, toy, toz, tdx, tdy, tdz, ray_v, ts_v, te_v,
          bpx, bpy, bpz, bgx, bgy, bgz, bdl):
        wid = lax.axis_index("s") * 2 + lax.axis_index("c")
        base = wid * CH
        pltpu.sync_copy(ox_h, tox)
        pltpu.sync_copy(oy_h, toy)
        pltpu.sync_copy(oz_h, toz)
        pltpu.sync_copy(dx_h, tdx)
        pltpu.sync_copy(dy_h, tdy)
        pltpu.sync_copy(dz_h, tdz)
        pltpu.sync_copy(ray_h.at[pl.ds(base, CH)], ray_v)
        pltpu.sync_copy(ts_h.at[pl.ds(base, CH)], ts_v)
        pltpu.sync_copy(te_h.at[pl.ds(base, CH)], te_v)
        for h in range(2):
            def step(j, _, h=h):
                o = h * HB + j * _L
                ol = j * _L
                idx = ray_v[pl.ds(o, _L)]
                tsv = ts_v[pl.ds(o, _L)]
                tev = te_v[pl.ds(o, _L)]
                tm = (tsv + tev) * 0.5
                gox = plsc.load_gather(tox, [idx])
                goy = plsc.load_gather(toy, [idx])
                goz = plsc.load_gather(toz, [idx])
                gdx = plsc.load_gather(tdx, [idx])
                gdy = plsc.load_gather(tdy, [idx])
                gdz = plsc.load_gather(tdz, [idx])
                bpx[pl.ds(ol, _L)] = gox + gdx * tm
                bpy[pl.ds(ol, _L)] = goy + gdy * tm
                bpz[pl.ds(ol, _L)] = goz + gdz * tm
                bgx[pl.ds(ol, _L)] = gdx
                bgy[pl.ds(ol, _L)] = gdy
                bgz[pl.ds(ol, _L)] = gdz
                bdl[pl.ds(ol, _L)] = tev - tsv
                return 0
            lax.fori_loop(0, HB // _L, step, 0)
            off = base + h * HB
            pltpu.sync_copy(bpx, px_h.at[pl.ds(off, HB)])
            pltpu.sync_copy(bpy, py_h.at[pl.ds(off, HB)])
            pltpu.sync_copy(bpz, pz_h.at[pl.ds(off, HB)])
            pltpu.sync_copy(bgx, gx_h.at[pl.ds(off, HB)])
            pltpu.sync_copy(bgy, gy_h.at[pl.ds(off, HB)])
            pltpu.sync_copy(bgz, gz_h.at[pl.ds(off, HB)])
            pltpu.sync_copy(bdl, dl_h.at[pl.ds(off, HB)])

    return k(ox, oy, oz, dx, dy, dz, ray, ts, te)


_BS = 2048


def _mlp_body(px, py, pz, gx, gy, gz, dl,
              W0tp, b0c, W1t, b1c, W2t, b2c, Wdc, bds, Wft, bfc, Wce, bcc,
              ro, go, bo, so, ao):
    z = jnp.zeros((1, _BS), f32)
    x8 = jnp.concatenate(
        [px[...], py[...], pz[...], z, gx[...], gy[...], gz[...], z], axis=0)
    h = jnp.maximum(
        jnp.dot(W0tp[...], x8, preferred_element_type=f32) + b0c[...], 0.0)
    h = jnp.maximum(
        jnp.dot(W1t[...], h, preferred_element_type=f32) + b1c[...], 0.0)
    h2 = jnp.maximum(
        jnp.dot(W2t[...], h, preferred_element_type=f32) + b2c[...], 0.0)
    dens = jnp.sum(h2 * Wdc[...], axis=0, keepdims=True) + bds[...]
    feat = jnp.maximum(
        jnp.dot(Wft[...], h2, preferred_element_type=f32) + bfc[...], 0.0)
    fe = jnp.concatenate([feat, x8], axis=0)
    rgb8 = jnp.dot(Wce[...], fe, preferred_element_type=f32) + bcc[...]
    rgb8 = 1.0 / (1.0 + jnp.exp(-rgb8))
    s = jnp.maximum(dens, 0.0) * dl[...]
    a = 1.0 - jnp.exp(-s)
    ro[...] = rgb8[0:1]
    go[...] = rgb8[1:2]
    bo[...] = rgb8[2:3]
    so[...] = s
    ao[...] = a


def _tc_mlp(px, py, pz, gx, gy, gz, dl, weights, S):
    NB = S // _BS
    row = pl.BlockSpec((1, _BS), lambda i: (0, i))
    full = lambda w: pl.BlockSpec(w.shape, lambda i: tuple(0 for _ in w.shape))
    in_specs = [row] * 7 + [full(w) for w in weights]
    out_specs = [row] * 5
    outs = [jax.ShapeDtypeStruct((1, S), f32) for _ in range(5)]
    fn = pl.pallas_call(
        _mlp_body,
        grid=(NB,),
        in_specs=in_specs,
        out_specs=out_specs,
        out_shape=outs,
    )
    rows = [a.reshape(1, S) for a in (px, py, pz, gx, gy, gz, dl)]
    return fn(*rows, *weights)


@functools.partial(jax.jit, static_argnames=("S", "n_rays"))
def _sc_composite(s_a, a_a, r_a, g_a, b_a, ts, te, ray, *, S, n_rays):
    NW = 16
    CH = S // NW
    SUB = 2048
    NSUB = CH // SUB
    NV = SUB // _L
    NQ = 5
    MR = n_rays // NW
    mesh = plsc.VectorSubcoreMesh(
        core_axis_name="c", subcore_axis_name="s", num_cores=1)
    out = [jax.ShapeDtypeStruct((n_rays,), f32) for _ in range(NQ)]
    scratch = (
        [pltpu.VMEM((CH,), f32), pltpu.VMEM((CH,), i32)]
        + [pltpu.VMEM((SUB,), f32) for _ in range(6)]
        + [pltpu.VMEM((NQ * n_rays,), f32),
           pltpu.VMEM((_L,), i32),
           pltpu.VMEM((_L,), f32),
           pltpu.VMEM((NW, _L), f32),
           pltpu.VMEM((NQ * MR,), f32),
           pltpu.VMEM((NQ * MR,), f32),
           pltpu.VMEM_SHARED((NW, _L), f32),
           pltpu.VMEM_SHARED((NW, NQ * n_rays), f32)]
        + [pltpu.VMEM((MR,), f32) for _ in range(NQ)]
    )

    @functools.partial(pl.kernel, mesh=mesh, out_type=out, scratch_types=scratch)
    def k(s_h, a_h, r_h, g_h, b_h, ts_h, te_h, ray_h,
          cr_h, cg_h, cb_h, dp_h, op_h,
          sb_s, sb_ray, sub_a, sub_r, sub_g, sub_b, sub_ts, sub_te,
          acc, bnd, info_o, linfo, red, tslice, sh_info, sh_acc,
          ob0, ob1, ob2, ob3, ob4):
        wid = lax.axis_index("s") + NW * lax.axis_index("c")
        base = wid * CH
        it = _iota16()

        def zacc(i, _):
            acc[pl.ds(i * _L, _L)] = jnp.zeros((_L,), f32)
            return 0
        lax.fori_loop(0, (NQ * n_rays) // _L, zacc, 0)

        pltpu.sync_copy(s_h.at[pl.ds(base, CH)], sb_s)
        pltpu.sync_copy(ray_h.at[pl.ds(base, CH)], sb_ray)

        poff = jnp.maximum(base - _L, 0)
        pltpu.sync_copy(ray_h.at[pl.ds(poff, _L)], bnd)
        bv = bnd[...]
        prev_ray = jnp.max(jnp.where(it == (_L - 1), bv, -1))
        prev_ray = jnp.where(wid > 0, prev_ray, -1)

        def p1(j, carry):
            ce, runA = carry
            o = j * _L
            sv = sb_s[pl.ds(o, _L)]
            rv = sb_ray[pl.ds(o, _L)]
            rpv = plsc.load_gather(sb_ray, [jnp.maximum(it + (o - 1), 0)])
            rpv = jnp.where(it + o == 0, prev_ray, rpv)
            cs = plsc.cumsum(sv)
            excl = ce + cs - sv
            st = rv != rpv
            runA = jnp.maximum(runA, jnp.max(jnp.where(st, excl, -1.0)))
            ce = ce + jnp.max(cs)
            return ce, runA
        total, A = lax.fori_loop(0, CH // _L, p1, (f32(0.0), f32(-1.0)))
        has_f = jnp.where(A >= 0.0, f32(1.0), f32(0.0))
        lastv = plsc.load_gather(sb_ray, [_splat_i(CH - 1)])
        last_ray_f = jnp.max(lastv).astype(f32)

        iv = jnp.where(it == 0, total,
                       jnp.where(it == 1, A,
                                 jnp.where(it == 2, has_f,
                                           jnp.where(it == 3, last_ray_f,
                                                     f32(0.0)))))
        info_o[...] = iv
        pltpu.sync_copy(info_o, sh_info.at[wid])
        plsc.subcore_barrier()
        pltpu.sync_copy(sh_info, linfo)

        totals = plsc.load_gather(linfo, [it, _splat_i(0)])
        Avec = plsc.load_gather(linfo, [it, _splat_i(1)])
        hasv = plsc.load_gather(linfo, [it, _splat_i(2)])
        ecst = plsc.cumsum(totals) - totals
        offset = jnp.sum(jnp.where(it < wid, totals, 0.0))
        carry_start = jnp.max(
            jnp.where((it < wid) & (hasv > 0.0), ecst + Avec, -1.0))

        ce = offset
        ssr = carry_start
        qcs = [f32(0.0)] * NQ
        for hblk in range(NSUB):
            hoff = base + hblk * SUB
            pltpu.sync_copy(a_h.at[pl.ds(hoff, SUB)], sub_a)
            pltpu.sync_copy(r_h.at[pl.ds(hoff, SUB)], sub_r)
            pltpu.sync_copy(g_h.at[pl.ds(hoff, SUB)], sub_g)
            pltpu.sync_copy(b_h.at[pl.ds(hoff, SUB)], sub_b)
            pltpu.sync_copy(ts_h.at[pl.ds(hoff, SUB)], sub_ts)
            pltpu.sync_copy(te_h.at[pl.ds(hoff, SUB)], sub_te)

            def p2(j, carry, hblk=hblk):
                ce, ssr, qc0, qc1, qc2, qc3, qc4 = carry
                qc = [qc0, qc1, qc2, qc3, qc4]
                o = hblk * SUB + j * _L
                ol = j * _L
                sv = sb_s[pl.ds(o, _L)]
                rv = sb_ray[pl.ds(o, _L)]
                rpv = plsc.load_gather(sb_ray, [jnp.maximum(it + (o - 1), 0)])
                rpv = jnp.where(it + o == 0, prev_ray, rpv)
                rnv = plsc.load_gather(
                    sb_ray, [jnp.minimum(it + (o + 1), CH - 1)])
                rnv = jnp.where(it + o == CH - 1, -1, rnv)
                cs = plsc.cumsum(sv)
                excl = ce + cs - sv
                ce = ce + jnp.max(cs)
                st = rv != rpv
                en = rv != rnv
                cm = plsc.cummax(jnp.where(st, excl, -1.0))
                segstart = jnp.maximum(cm, ssr)
                ssr = jnp.max(segstart)
                pre = excl - segstart
                av = sub_a[pl.ds(ol, _L)]
                wv = jnp.exp(-pre) * av
                tsv = sub_ts[pl.ds(ol, _L)]
                tev = sub_te[pl.ds(ol, _L)]
                tm = (tsv + tev) * 0.5
                rr = sub_r[pl.ds(ol, _L)]
                gg = sub_g[pl.ds(ol, _L)]
                bb = sub_b[pl.ds(ol, _L)]
                en15 = jnp.max(jnp.where(it == (_L - 1),
                                         jnp.where(en, f32(1.0), f32(0.0)),
                                         f32(0.0)))
                qvals = [wv * rr, wv * gg, wv * bb, wv, wv * tm]
                nqc = []
                for qi in range(NQ):
                    qv = qvals[qi]
                    csq = plsc.cumsum(qv)
                    ecsq = csq - qv
                    bq = plsc.cummax(jnp.where(st, ecsq, -1.0))
                    si = jnp.where(bq < 0.0, csq + qc[qi], csq - bq)
                    plsc.addupdate_scatter(
                        acc, [rv + qi * n_rays], si, mask=en)
                    si15 = jnp.max(jnp.where(it == (_L - 1), si, -1.0))
                    nqc.append((1.0 - en15) * si15)
                return (ce, ssr, nqc[0], nqc[1], nqc[2], nqc[3], nqc[4])

            ce, ssr, *qcs = lax.fori_loop(
                0, NV, p2, (ce, ssr, qcs[0], qcs[1], qcs[2], qcs[3], qcs[4]))

        pltpu.sync_copy(acc, sh_acc.at[wid])
        plsc.subcore_barrier()

        def zred(i, _):
            red[pl.ds(i * _L, _L)] = jnp.zeros((_L,), f32)
            return 0
        lax.fori_loop(0, (NQ * MR) // _L, zred, 0)
        rb = wid * MR
        for t in range(NW):
            for qi in range(NQ):
                pltpu.sync_copy(sh_acc.at[t, pl.ds(qi * n_rays + rb, MR)],
                                tslice.at[pl.ds(qi * MR, MR)])

            def radd(i, _):
                o = i * _L
                red[pl.ds(o, _L)] = red[pl.ds(o, _L)] + tslice[pl.ds(o, _L)]
                return 0
            lax.fori_loop(0, (NQ * MR) // _L, radd, 0)

        def fin(i, _):
            o = i * _L
            opv = red[pl.ds(3 * MR + o, _L)]
            ob0[pl.ds(o, _L)] = red[pl.ds(0 * MR + o, _L)] + (1.0 - opv)
            ob1[pl.ds(o, _L)] = red[pl.ds(1 * MR + o, _L)] + (1.0 - opv)
            ob2[pl.ds(o, _L)] = red[pl.ds(2 * MR + o, _L)] + (1.0 - opv)
            ob3[pl.ds(o, _L)] = red[pl.ds(4 * MR + o, _L)]
            ob4[pl.ds(o, _L)] = opv
            return 0
        lax.fori_loop(0, MR // _L, fin, 0)
        pltpu.sync_copy(ob0, cr_h.at[pl.ds(rb, MR)])
        pltpu.sync_copy(ob1, cg_h.at[pl.ds(rb, MR)])
        pltpu.sync_copy(ob2, cb_h.at[pl.ds(rb, MR)])
        pltpu.sync_copy(ob3, dp_h.at[pl.ds(rb, MR)])
        pltpu.sync_copy(ob4, op_h.at[pl.ds(rb, MR)])

    return k(s_a, a_a, r_a, g_a, b_a, ts, te, ray)


def kernel(xyz, viewdirs, ray_indices, t_start, t_ends,
           W0, b0, W1, b1, W2, b2, Wd, bd, Wf, bf, Wc, bc):
    Bb, R, _ = xyz.shape
    n_rays = Bb * R
    S = ray_indices.shape[0]
    ro = xyz.reshape(-1, 3)
    vd = viewdirs.reshape(-1, 3)
    ray = ray_indices.astype(i32)
    ts = t_start.astype(f32)
    te = t_ends.astype(f32)

    px, py, pz, gx, gy, gz, dl = _sc_gather(
        ro[:, 0], ro[:, 1], ro[:, 2], vd[:, 0], vd[:, 1], vd[:, 2],
        ray, ts, te, S=S, n_rays=n_rays)

    W0tp = jnp.concatenate([W0.T, jnp.zeros((W0.shape[1], 5), f32)], axis=1)
    Wce = jnp.zeros((8, 136), f32)
    Wce = Wce.at[:3, :128].set(Wc[:128, :].T)
    Wce = Wce.at[:3, 132:135].set(Wc[128:131, :].T)
    bcc = jnp.concatenate([bc, jnp.zeros((5,), f32)])[:, None]
    weights = [W0tp, b0[:, None], W1.T, b1[:, None], W2.T, b2[:, None],
               Wd, bd.reshape(1, 1), Wf.T, bf[:, None], Wce, bcc]

    r_, g_, b_, s_, a_ = _tc_mlp(px, py, pz, gx, gy, gz, dl, weights, S)

    cr, cg, cb, dp, op = _sc_composite(
        s_.reshape(S), a_.reshape(S), r_.reshape(S), g_.reshape(S),
        b_.reshape(S), ts, te, ray, S=S, n_rays=n_rays)

    colors = jnp.stack([cr, cg, cb], axis=-1).reshape(Bb, R, 3)
    depths = dp.reshape(Bb, R, 1)
    opac = op.reshape(Bb, R, 1)
    return colors, depths, opac


# speedup vs baseline: 16.0761x; 16.0761x over previous
import functools

import jax
import jax.numpy as jnp
from jax import lax
from jax.experimental import pallas as pl
from jax.experimental.pallas import tpu as pltpu
from jax.experimental.pallas import tpu_sc as plsc

_L = 16
f32 = jnp.float32
i32 = jnp.int32


def _iota16():
    return lax.iota(i32, _L)


def _splat_i(v):
    return jnp.zeros((_L,), i32) + v


@functools.partial(jax.jit, static_argnames=("S", "n_rays"))
def _sc_gather(ox, oy, oz, dx, dy, dz, ray, ts, te, *, S, n_rays):
    NW = 32
    CH = S // NW
    HB = CH // 2
    mesh = plsc.VectorSubcoreMesh(core_axis_name="c", subcore_axis_name="s")
    out = [jax.ShapeDtypeStruct((S,), f32) for _ in range(7)]
    scratch = (
        [pltpu.VMEM((n_rays,), f32) for _ in range(6)]
        + [pltpu.VMEM((CH,), i32), pltpu.VMEM((CH,), f32), pltpu.VMEM((CH,), f32)]
        + [pltpu.VMEM((HB,), f32) for _ in range(7)]
    )

    @functools.partial(pl.kernel, mesh=mesh, out_type=out, scratch_types=scratch,
                       compiler_params=pltpu.CompilerParams(needs_layout_passes=False))
    def k(ox_h, oy_h, oz_h, dx_h, dy_h, dz_h, ray_h, ts_h, te_h,
          px_h, py_h, pz_h, gx_h, gy_h, gz_h, dl_h,
          tox, toy, toz, tdx, tdy, tdz, ray_v, ts_v, te_v,
          bpx, bpy, bpz, bgx, bgy, bgz, bdl):
        wid = lax.axis_index("s") * 2 + lax.axis_index("c")
        base = wid * CH
        pltpu.sync_copy(ox_h, tox)
        pltpu.sync_copy(oy_h, toy)
        pltpu.sync_copy(oz_h, toz)
        pltpu.sync_copy(dx_h, tdx)
        pltpu.sync_copy(dy_h, tdy)
        pltpu.sync_copy(dz_h, tdz)
        pltpu.sync_copy(ray_h.at[pl.ds(base, CH)], ray_v)
        pltpu.sync_copy(ts_h.at[pl.ds(base, CH)], ts_v)
        pltpu.sync_copy(te_h.at[pl.ds(base, CH)], te_v)
        for h in range(2):
            def step(j, _, h=h):
                o = h * HB + j * _L
                ol = j * _L
                idx = ray_v[pl.ds(o, _L)]
                tsv = ts_v[pl.ds(o, _L)]
                tev = te_v[pl.ds(o, _L)]
                tm = (tsv + tev) * 0.5
                gox = plsc.load_gather(tox, [idx])
                goy = plsc.load_gather(toy, [idx])
                goz = plsc.load_gather(toz, [idx])
                gdx = plsc.load_gather(tdx, [idx])
                gdy = plsc.load_gather(tdy, [idx])
                gdz = plsc.load_gather(tdz, [idx])
                bpx[pl.ds(ol, _L)] = gox + gdx * tm
                bpy[pl.ds(ol, _L)] = goy + gdy * tm
                bpz[pl.ds(ol, _L)] = goz + gdz * tm
                bgx[pl.ds(ol, _L)] = gdx
                bgy[pl.ds(ol, _L)] = gdy
                bgz[pl.ds(ol, _L)] = gdz
                bdl[pl.ds(ol, _L)] = tev - tsv
                return 0
            lax.fori_loop(0, HB // _L, step, 0)
            off = base + h * HB
            pltpu.sync_copy(bpx, px_h.at[pl.ds(off, HB)])
            pltpu.sync_copy(bpy, py_h.at[pl.ds(off, HB)])
            pltpu.sync_copy(bpz, pz_h.at[pl.ds(off, HB)])
            pltpu.sync_copy(bgx, gx_h.at[pl.ds(off, HB)])
            pltpu.sync_copy(bgy, gy_h.at[pl.ds(off, HB)])
            pltpu.sync_copy(bgz, gz_h.at[pl.ds(off, HB)])
            pltpu.sync_copy(bdl, dl_h.at[pl.ds(off, HB)])

    return k(ox, oy, oz, dx, dy, dz, ray, ts, te)


_BS = 2048


def _mlp_body(px, py, pz, gx, gy, gz, dl,
              W0tp, b0c, W1t, b1c, W2t, b2c, Wdc, bds, Wft, bfc, Wce, bcc,
              ro, go, bo, so, ao):
    z = jnp.zeros((1, _BS), f32)
    x8 = jnp.concatenate(
        [px[...], py[...], pz[...], z, gx[...], gy[...], gz[...], z], axis=0)
    h = jnp.maximum(
        jnp.dot(W0tp[...], x8, preferred_element_type=f32) + b0c[...], 0.0)
    h = jnp.maximum(
        jnp.dot(W1t[...], h, preferred_element_type=f32) + b1c[...], 0.0)
    h2 = jnp.maximum(
        jnp.dot(W2t[...], h, preferred_element_type=f32) + b2c[...], 0.0)
    dens = jnp.sum(h2 * Wdc[...], axis=0, keepdims=True) + bds[...]
    feat = jnp.maximum(
        jnp.dot(Wft[...], h2, preferred_element_type=f32) + bfc[...], 0.0)
    fe = jnp.concatenate([feat, x8], axis=0)
    rgb8 = jnp.dot(Wce[...], fe, preferred_element_type=f32) + bcc[...]
    rgb8 = 1.0 / (1.0 + jnp.exp(-rgb8))
    s = jnp.maximum(dens, 0.0) * dl[...]
    a = 1.0 - jnp.exp(-s)
    ro[...] = rgb8[0:1]
    go[...] = rgb8[1:2]
    bo[...] = rgb8[2:3]
    so[...] = s
    ao[...] = a


def _tc_mlp(px, py, pz, gx, gy, gz, dl, weights, S):
    NB = S // _BS
    row = pl.BlockSpec((1, _BS), lambda i: (0, i))
    full = lambda w: pl.BlockSpec(w.shape, lambda i: tuple(0 for _ in w.shape))
    in_specs = [row] * 7 + [full(w) for w in weights]
    out_specs = [row] * 5
    outs = [jax.ShapeDtypeStruct((1, S), f32) for _ in range(5)]
    fn = pl.pallas_call(
        _mlp_body,
        grid=(NB,),
        in_specs=in_specs,
        out_specs=out_specs,
        out_shape=outs,
    )
    rows = [a.reshape(1, S) for a in (px, py, pz, gx, gy, gz, dl)]
    return fn(*rows, *weights)


@functools.partial(jax.jit, static_argnames=("S", "n_rays"))
def _sc_composite(s_a, a_a, r_a, g_a, b_a, ts, te, ray, *, S, n_rays):
    NW = 16
    CH = S // NW
    SUB = 2048
    NSUB = CH // SUB
    NV = SUB // _L
    NQ = 5
    MR = n_rays // NW
    mesh = plsc.VectorSubcoreMesh(
        core_axis_name="c", subcore_axis_name="s", num_cores=1)
    NR = 4
    RR = n_rays // NR
    out = [jax.ShapeDtypeStruct((n_rays,), f32) for _ in range(NQ)]
    scratch = (
        [pltpu.VMEM((CH,), i32)]
        + [pltpu.VMEM((SUB,), f32) for _ in range(7)]
        + [pltpu.VMEM((NQ * n_rays,), f32),
           pltpu.VMEM((_L,), i32),
           pltpu.VMEM((_L,), f32),
           pltpu.VMEM((NW, _L), f32),
           pltpu.VMEM((NQ * MR,), f32),
           pltpu.VMEM((NQ * MR,), f32),
           pltpu.VMEM_SHARED((NW, _L), f32),
           pltpu.VMEM_SHARED((NW, NQ * RR), f32)]
        + [pltpu.VMEM((MR,), f32) for _ in range(NQ)]
    )

    @functools.partial(pl.kernel, mesh=mesh, out_type=out, scratch_types=scratch,
                       compiler_params=pltpu.CompilerParams(needs_layout_passes=False))
    def k(s_h, a_h, r_h, g_h, b_h, ts_h, te_h, ray_h,
          cr_h, cg_h, cb_h, dp_h, op_h,
          sb_ray, sub_s, sub_a, sub_r, sub_g, sub_b, sub_ts, sub_te,
          acc, bnd, info_o, linfo, red, tslice, sh_info, sh_stage,
          ob0, ob1, ob2, ob3, ob4):
        wid = lax.axis_index("s") + NW * lax.axis_index("c")
        base = wid * CH
        it = _iota16()

        def zacc(i, _):
            acc[pl.ds(i * _L, _L)] = jnp.zeros((_L,), f32)
            return 0
        lax.fori_loop(0, (NQ * n_rays) // _L, zacc, 0)

        pltpu.sync_copy(ray_h.at[pl.ds(base, CH)], sb_ray)

        poff = pl.multiple_of(jnp.maximum(base - _L, 0), _L)
        pltpu.sync_copy(ray_h.at[pl.ds(poff, _L)], bnd)
        bv = bnd[...]
        prev_ray = jnp.max(jnp.where(it == (_L - 1), bv, -1))
        prev_ray = jnp.where(wid > 0, prev_ray, -1)

        ce1 = f32(0.0)
        runA1 = f32(-1.0)
        for hblk in range(NSUB):
            hoff = base + hblk * SUB
            pltpu.sync_copy(s_h.at[pl.ds(hoff, SUB)], sub_s)

            def p1(j, carry, hblk=hblk):
                ce, runA = carry
                o = hblk * SUB + j * _L
                ol = j * _L
                sv = sub_s[pl.ds(ol, _L)]
                rv = sb_ray[pl.ds(o, _L)]
                rpv = plsc.load_gather(sb_ray, [jnp.maximum(it + (o - 1), 0)])
                rpv = jnp.where(it + o == 0, prev_ray, rpv)
                cs = plsc.cumsum(sv)
                excl = ce + cs - sv
                st = rv != rpv
                runA = jnp.maximum(runA, jnp.max(jnp.where(st, excl, -1.0)))
                ce = ce + jnp.max(cs)
                return ce, runA
            ce1, runA1 = lax.fori_loop(0, NV, p1, (ce1, runA1))
        total, A = ce1, runA1
        has_f = jnp.where(A >= 0.0, f32(1.0), f32(0.0))
        lastv = plsc.load_gather(sb_ray, [_splat_i(CH - 1)])
        last_ray_f = jnp.max(lastv).astype(f32)

        iv = jnp.where(it == 0, total,
                       jnp.where(it == 1, A,
                                 jnp.where(it == 2, has_f,
                                           jnp.where(it == 3, last_ray_f,
                                                     f32(0.0)))))
        info_o[...] = iv
        pltpu.sync_copy(info_o, sh_info.at[wid])
        plsc.subcore_barrier()
        pltpu.sync_copy(sh_info, linfo)

        totals = plsc.load_gather(linfo, [it, _splat_i(0)])
        Avec = plsc.load_gather(linfo, [it, _splat_i(1)])
        hasv = plsc.load_gather(linfo, [it, _splat_i(2)])
        ecst = plsc.cumsum(totals) - totals
        offset = jnp.sum(jnp.where(it < wid, totals, 0.0))
        carry_start = jnp.max(
            jnp.where((it < wid) & (hasv > 0.0), ecst + Avec, -1.0))

        ce = offset
        ssr = carry_start
        qcs = [f32(0.0)] * NQ
        for hblk in range(NSUB):
            hoff = base + hblk * SUB
            pltpu.sync_copy(s_h.at[pl.ds(hoff, SUB)], sub_s)
            pltpu.sync_copy(a_h.at[pl.ds(hoff, SUB)], sub_a)
            pltpu.sync_copy(r_h.at[pl.ds(hoff, SUB)], sub_r)
            pltpu.sync_copy(g_h.at[pl.ds(hoff, SUB)], sub_g)
            pltpu.sync_copy(b_h.at[pl.ds(hoff, SUB)], sub_b)
            pltpu.sync_copy(ts_h.at[pl.ds(hoff, SUB)], sub_ts)
            pltpu.sync_copy(te_h.at[pl.ds(hoff, SUB)], sub_te)

            def p2(j, carry, hblk=hblk):
                ce, ssr, qc0, qc1, qc2, qc3, qc4 = carry
                qc = [qc0, qc1, qc2, qc3, qc4]
                o = hblk * SUB + j * _L
                ol = j * _L
                sv = sub_s[pl.ds(ol, _L)]
                rv = sb_ray[pl.ds(o, _L)]
                rpv = plsc.load_gather(sb_ray, [jnp.maximum(it + (o - 1), 0)])
                rpv = jnp.where(it + o == 0, prev_ray, rpv)
                rnv = plsc.load_gather(
                    sb_ray, [jnp.minimum(it + (o + 1), CH - 1)])
                rnv = jnp.where(it + o == CH - 1, -1, rnv)
                cs = plsc.cumsum(sv)
                excl = ce + cs - sv
                ce = ce + jnp.max(cs)
                st = rv != rpv
                en = rv != rnv
                cm = plsc.cummax(jnp.where(st, excl, -1.0))
                segstart = jnp.maximum(cm, ssr)
                ssr = jnp.max(segstart)
                pre = excl - segstart
                av = sub_a[pl.ds(ol, _L)]
                wv = jnp.exp(-pre) * av
                tsv = sub_ts[pl.ds(ol, _L)]
                tev = sub_te[pl.ds(ol, _L)]
                tm = (tsv + tev) * 0.5
                rr = sub_r[pl.ds(ol, _L)]
                gg = sub_g[pl.ds(ol, _L)]
                bb = sub_b[pl.ds(ol, _L)]
                en15 = jnp.max(jnp.where(it == (_L - 1),
                                         jnp.where(en, f32(1.0), f32(0.0)),
                                         f32(0.0)))
                qvals = [wv * rr, wv * gg, wv * bb, wv, wv * tm]
                nqc = []
                for qi in range(NQ):
                    qv = qvals[qi]
                    csq = plsc.cumsum(qv)
                    ecsq = csq - qv
                    bq = plsc.cummax(jnp.where(st, ecsq, -1.0))
                    si = jnp.where(bq < 0.0, csq + qc[qi], csq - bq)
                    plsc.addupdate_scatter(
                        acc, [rv + qi * n_rays], si, mask=en)
                    si15 = jnp.max(jnp.where(it == (_L - 1), si, -1.0))
                    nqc.append((1.0 - en15) * si15)
                return (ce, ssr, nqc[0], nqc[1], nqc[2], nqc[3], nqc[4])

            ce, ssr, *qcs = lax.fori_loop(
                0, NV, p2, (ce, ssr, qcs[0], qcs[1], qcs[2], qcs[3], qcs[4]))

        rb = pl.multiple_of(wid * MR, MR)
        for rnd in range(NR):
            for qi in range(NQ):
                pltpu.sync_copy(
                    acc.at[pl.ds(qi * n_rays + rnd * RR, RR)],
                    sh_stage.at[wid, pl.ds(qi * RR, RR)])
            plsc.subcore_barrier()

            @pl.when((wid >= rnd * (NW // NR)) & (wid < (rnd + 1) * (NW // NR)))
            def _(rnd=rnd):
                loff = pl.multiple_of((wid - rnd * (NW // NR)) * MR, MR)

                def zred(i, _):
                    red[pl.ds(i * _L, _L)] = jnp.zeros((_L,), f32)
                    return 0
                lax.fori_loop(0, (NQ * MR) // _L, zred, 0)

                def tred(t, _):
                    for qi in range(NQ):
                        pltpu.sync_copy(
                            sh_stage.at[t, pl.ds(qi * RR + loff, MR)],
                            tslice.at[pl.ds(qi * MR, MR)])

                    def radd(i, _):
                        o = i * _L
                        red[pl.ds(o, _L)] = (red[pl.ds(o, _L)]
                                             + tslice[pl.ds(o, _L)])
                        return 0
                    lax.fori_loop(0, (NQ * MR) // _L, radd, 0)
                    return 0
                lax.fori_loop(0, NW, tred, 0)

                def fin(i, _):
                    o = i * _L
                    opv = red[pl.ds(3 * MR + o, _L)]
                    ob0[pl.ds(o, _L)] = red[pl.ds(0 * MR + o, _L)] + (1.0 - opv)
                    ob1[pl.ds(o, _L)] = red[pl.ds(1 * MR + o, _L)] + (1.0 - opv)
                    ob2[pl.ds(o, _L)] = red[pl.ds(2 * MR + o, _L)] + (1.0 - opv)
                    ob3[pl.ds(o, _L)] = red[pl.ds(4 * MR + o, _L)]
                    ob4[pl.ds(o, _L)] = opv
                    return 0
                lax.fori_loop(0, MR // _L, fin, 0)
                pltpu.sync_copy(ob0, cr_h.at[pl.ds(rb, MR)])
                pltpu.sync_copy(ob1, cg_h.at[pl.ds(rb, MR)])
                pltpu.sync_copy(ob2, cb_h.at[pl.ds(rb, MR)])
                pltpu.sync_copy(ob3, dp_h.at[pl.ds(rb, MR)])
                pltpu.sync_copy(ob4, op_h.at[pl.ds(rb, MR)])
            plsc.subcore_barrier()

    return k(s_a, a_a, r_a, g_a, b_a, ts, te, ray)


def kernel(xyz, viewdirs, ray_indices, t_start, t_ends,
           W0, b0, W1, b1, W2, b2, Wd, bd, Wf, bf, Wc, bc):
    Bb, R, _ = xyz.shape
    n_rays = Bb * R
    S = ray_indices.shape[0]
    ro = xyz.reshape(-1, 3)
    vd = viewdirs.reshape(-1, 3)
    ray = ray_indices.astype(i32)
    ts = t_start.astype(f32)
    te = t_ends.astype(f32)

    px, py, pz, gx, gy, gz, dl = _sc_gather(
        ro[:, 0], ro[:, 1], ro[:, 2], vd[:, 0], vd[:, 1], vd[:, 2],
        ray, ts, te, S=S, n_rays=n_rays)

    W0tp = jnp.concatenate([W0.T, jnp.zeros((W0.shape[1], 5), f32)], axis=1)
    Wce = jnp.zeros((8, 136), f32)
    Wce = Wce.at[:3, :128].set(Wc[:128, :].T)
    Wce = Wce.at[:3, 132:135].set(Wc[128:131, :].T)
    bcc = jnp.concatenate([bc, jnp.zeros((5,), f32)])[:, None]
    weights = [W0tp, b0[:, None], W1.T, b1[:, None], W2.T, b2[:, None],
               Wd, bd.reshape(1, 1), Wf.T, bf[:, None], Wce, bcc]

    r_, g_, b_, s_, a_ = _tc_mlp(px, py, pz, gx, gy, gz, dl, weights, S)

    cr, cg, cb, dp, op = _sc_composite(
        s_.reshape(S), a_.reshape(S), r_.reshape(S), g_.reshape(S),
        b_.reshape(S), ts, te, ray, S=S, n_rays=n_rays)

    colors = jnp.stack([cr, cg, cb], axis=-1).reshape(Bb, R, 3)
    depths = dp.reshape(Bb, R, 1)
    opac = op.reshape(Bb, R, 1)
    return colors, depths, opac
